# baseline (device time: 19275 ns/iter reference)
import jax
import jax.numpy as jnp
from jax import lax
from jax.experimental import pallas as pl
from jax.experimental.pallas import tpu as pltpu

N_DEV = 4


def kernel(x, router_W, route_idx, expert_W, shared_W):
    n_tok, d_model = x.shape
    d_out = shared_W.shape[1]
    n_local = expert_W.shape[0]
    chunk = n_tok // N_DEV

    def body(x_ref, rw_ref, idx_ref, ew_ref, sw_ref, out_ref,
             psel_buf, ewb, swb, sbuf, rs_buf, ag_buf, r_buf,
             send_sems, recv_sems):
        my = lax.axis_index("i")

        barrier_sem = pltpu.get_barrier_semaphore()
        for dq in range(1, N_DEV):
            pl.semaphore_signal(
                barrier_sem, inc=1,
                device_id=(lax.rem(my + dq, N_DEV),),
                device_id_type=pl.DeviceIdType.MESH,
            )
        pl.semaphore_wait(barrier_sem, N_DEV - 1)

        xv = x_ref[:, :]

        for j in range(n_local):
            ewb[j, :, :] = ew_ref[j, :, :].astype(jnp.bfloat16)
        swb[:, :] = sw_ref[:, :].astype(jnp.bfloat16)

        scores = jnp.dot(xv, rw_ref[:, :], preferred_element_type=jnp.float32)
        m = jnp.max(scores, axis=-1, keepdims=True)
        p = jnp.exp(scores - m)
        probs = p / jnp.sum(p, axis=-1, keepdims=True)
        eids = lax.broadcasted_iota(jnp.int32, scores.shape, 1)
        psel_buf[:, :] = jnp.sum(
            jnp.where(eids == idx_ref[:, :], probs, 0.0),
            axis=-1, keepdims=True,
        )

        def chunk_partial(q):
            qs = q * chunk
            xq = x_ref[pl.ds(qs, chunk), :]
            iq = idx_ref[pl.ds(qs, chunk), :]
            pq = psel_buf[pl.ds(qs, chunk), :]
            acc = jnp.zeros((chunk, d_out), dtype=jnp.float32)
            for j in range(n_local):
                coeff = jnp.where(iq == my * n_local + j, pq, 0.0)
                acc = acc + jnp.dot(
                    (xq * coeff).astype(jnp.bfloat16), ewb[j, :, :],
                    preferred_element_type=jnp.float32,
                )
            return acc

        rs_rdmas = []
        for dq in (2, 1, 3):
            q = lax.rem(my + dq, N_DEV)
            slot = (N_DEV - 1) - dq
            sbuf[slot, :, :] = chunk_partial(q).astype(jnp.bfloat16)
            rdma = pltpu.make_async_remote_copy(
                src_ref=sbuf.at[slot],
                dst_ref=rs_buf.at[slot],
                send_sem=send_sems.at[0, slot],
                recv_sem=recv_sems.at[0, slot],
                device_id=(q,),
                device_id_type=pl.DeviceIdType.MESH,
            )
            rdma.start()
            rs_rdmas.append(rdma)

        own = chunk_partial(my)
        shared = jnp.dot(
            xv.astype(jnp.bfloat16), swb[:, :],
            preferred_element_type=jnp.float32,
        )
        out_ref[:, :] = shared

        for rdma in rs_rdmas:
            rdma.wait_recv()

        r_f32 = (
            own
            + rs_buf[0, :, :].astype(jnp.float32)
            + rs_buf[1, :, :].astype(jnp.float32)
            + rs_buf[2, :, :].astype(jnp.float32)
        )
        r_buf[:, :] = r_f32.astype(jnp.bfloat16)

        ag_rdmas = []
        for dq in (2, 1, 3):
            q = lax.rem(my + dq, N_DEV)
            slot = (N_DEV - 1) - dq
            rdma = pltpu.make_async_remote_copy(
                src_ref=r_buf,
                dst_ref=ag_buf.at[slot],
                send_sem=send_sems.at[1, slot],
                recv_sem=recv_sems.at[1, slot],
                device_id=(q,),
                device_id_type=pl.DeviceIdType.MESH,
            )
            rdma.start()
            ag_rdmas.append(rdma)

        out_ref[pl.ds(my * chunk, chunk), :] = (
            out_ref[pl.ds(my * chunk, chunk), :] + r_f32
        )

        for rdma in ag_rdmas:
            rdma.wait_recv()
        for r in range(N_DEV - 1):
            s = lax.rem(my + r + 1, N_DEV)
            out_ref[pl.ds(s * chunk, chunk), :] = (
                out_ref[pl.ds(s * chunk, chunk), :]
                + ag_buf[r, :, :].astype(jnp.float32)
            )

        for rdma in rs_rdmas + ag_rdmas:
            rdma.wait_send()

    return pl.pallas_call(
        body,
        out_shape=jax.ShapeDtypeStruct((n_tok, d_out), jnp.float32),
        in_specs=[
            pl.BlockSpec(memory_space=pltpu.VMEM),
            pl.BlockSpec(memory_space=pltpu.VMEM),
            pl.BlockSpec(memory_space=pltpu.VMEM),
            pl.BlockSpec(memory_space=pltpu.VMEM),
            pl.BlockSpec(memory_space=pltpu.VMEM),
        ],
        out_specs=pl.BlockSpec(memory_space=pltpu.VMEM),
        scratch_shapes=[
            pltpu.VMEM((n_tok, 1), jnp.float32),
            pltpu.VMEM((n_local, d_model, d_out), jnp.bfloat16),
            pltpu.VMEM((d_model, d_out), jnp.bfloat16),
            pltpu.VMEM((N_DEV - 1, chunk, d_out), jnp.bfloat16),
            pltpu.VMEM((N_DEV - 1, chunk, d_out), jnp.bfloat16),
            pltpu.VMEM((N_DEV - 1, chunk, d_out), jnp.bfloat16),
            pltpu.VMEM((chunk, d_out), jnp.bfloat16),
            pltpu.SemaphoreType.DMA((2, N_DEV - 1)),
            pltpu.SemaphoreType.DMA((2, N_DEV - 1)),
        ],
        compiler_params=pltpu.CompilerParams(collective_id=0),
    )(x, router_W, route_idx, expert_W, shared_W)


# device time: 17925 ns/iter; 1.0753x vs baseline; 1.0753x over previous
import os

import jax
import jax.numpy as jnp
from jax import lax
from jax.experimental import pallas as pl
from jax.experimental.pallas import tpu as pltpu

N_DEV = 4

_ABLATE = os.environ.get("ABLATE", "")
_DO_COMM = _ABLATE not in ("compute", "all", "nobar")
_DO_MATH = _ABLATE not in ("comm", "all", "nobar")
_DO_BARRIER = _ABLATE != "nobar"


def kernel(x, router_W, route_idx, expert_W, shared_W):
    n_tok, d_model = x.shape
    d_out = shared_W.shape[1]
    n_local = expert_W.shape[0]
    chunk = n_tok // N_DEV
    half = d_out // 2

    def body(x_ref, rw_ref, idx_ref, ew_ref, sw_ref, out_ref,
             psel_buf, ewv, swv, sbuf, rs_buf, r_buf,
             copy_sems, send_sems, recv_sems):
        my = lax.axis_index("i")

        ew_copy = pltpu.make_async_copy(ew_ref, ewv, copy_sems.at[0])
        sw_copy = pltpu.make_async_copy(sw_ref, swv, copy_sems.at[1])
        ew_copy.start()
        sw_copy.start()

        if _DO_BARRIER:
            barrier_sem = pltpu.get_barrier_semaphore()
            for dq in (1, N_DEV - 1):
                pl.semaphore_signal(
                    barrier_sem, inc=1,
                    device_id=(lax.rem(my + dq, N_DEV),),
                    device_id_type=pl.DeviceIdType.MESH,
                )
            pl.semaphore_wait(barrier_sem, 2)

        xv = x_ref[:, :]

        scores = jnp.dot(xv, rw_ref[:, :], preferred_element_type=jnp.float32)
        m = jnp.max(scores, axis=-1, keepdims=True)
        p = jnp.exp(scores - m)
        probs = p / jnp.sum(p, axis=-1, keepdims=True)
        eids = lax.broadcasted_iota(jnp.int32, scores.shape, 1)
        psel_buf[:, :] = jnp.sum(
            jnp.where(eids == idx_ref[:, :], probs, 0.0),
            axis=-1, keepdims=True,
        )

        def chunk_partial(q):
            qs = q * chunk
            xq = x_ref[pl.ds(qs, chunk), :]
            iq = idx_ref[pl.ds(qs, chunk), :]
            pq = psel_buf[pl.ds(qs, chunk), :]
            acc = jnp.zeros((chunk, d_out), dtype=jnp.float32)
            for j in range(n_local):
                if not _DO_MATH:
                    break
                coeff = jnp.where(iq == my * n_local + j, pq, 0.0)
                acc = acc + jnp.dot(
                    xq * coeff, ewv[j, :, :],
                    preferred_element_type=jnp.float32,
                )
            return acc

        def shared_chunk(q):
            xq = x_ref[pl.ds(q * chunk, chunk), :]
            if not _DO_MATH:
                return jnp.zeros((chunk, d_out), dtype=jnp.float32)
            return jnp.dot(
                xq, swv[:, :], preferred_element_type=jnp.float32
            )

        def exchange(src_ref_h, dst_ref_h, phase, h, slot, q, start=True):
            rdma = pltpu.make_async_remote_copy(
                src_ref=src_ref_h,
                dst_ref=dst_ref_h,
                send_sem=send_sems.at[phase, h, slot],
                recv_sem=recv_sems.at[phase, h, slot],
                device_id=(q,),
                device_id_type=pl.DeviceIdType.MESH,
            )
            if _DO_COMM and start:
                rdma.start()
            return rdma

        ew_copy.wait()
        rs_rdmas = [[], []]
        for dq in (2, 1, 3):
            q = lax.rem(my + dq, N_DEV)
            slot = (N_DEV - 1) - dq
            pc = chunk_partial(q)
            for h in range(2):
                sbuf[h, slot, :, :] = (
                    pc[:, h * half:(h + 1) * half].astype(jnp.bfloat16)
                )
                rs_rdmas[h].append(
                    exchange(sbuf.at[h, slot], rs_buf.at[h, slot],
                             0, h, slot, q)
                )

        own = chunk_partial(my)
        sw_copy.wait()
        sh_my = shared_chunk(my)

        ag_rdmas = [[], []]
        for h in range(2):
            if _DO_COMM:
                for rdma in rs_rdmas[h]:
                    rdma.wait_recv()
            fin = (
                sh_my[:, h * half:(h + 1) * half]
                + own[:, h * half:(h + 1) * half]
                + rs_buf[h, 0, :, :].astype(jnp.float32)
                + rs_buf[h, 1, :, :].astype(jnp.float32)
                + rs_buf[h, 2, :, :].astype(jnp.float32)
            ).astype(jnp.bfloat16)
            r_buf[h, :, :] = fin
            out_ref[pl.ds(my * chunk, chunk), pl.ds(h * half, half)] = fin
            for dq in (2, 1, 3):
                q = lax.rem(my + dq, N_DEV)
                slot = (N_DEV - 1) - dq
                ag_rdmas[h].append(
                    exchange(
                        r_buf.at[h],
                        out_ref.at[pl.ds(my * chunk, chunk),
                                   pl.ds(h * half, half)],
                        1, h, slot, q,
                    )
                )

        if _DO_COMM:
            for h in range(2):
                for r in range(N_DEV - 1):
                    s = lax.rem(my + r + 1, N_DEV)
                    exchange(
                        r_buf.at[h],
                        out_ref.at[pl.ds(s * chunk, chunk),
                                   pl.ds(h * half, half)],
                        1, h, r, s, start=False,
                    ).wait_recv()

        if _DO_COMM:
            for h in range(2):
                for rdma in rs_rdmas[h] + ag_rdmas[h]:
                    rdma.wait_send()

    return pl.pallas_call(
        body,
        out_shape=jax.ShapeDtypeStruct((n_tok, d_out), jnp.bfloat16),
        in_specs=[
            pl.BlockSpec(memory_space=pltpu.VMEM),
            pl.BlockSpec(memory_space=pltpu.VMEM),
            pl.BlockSpec(memory_space=pltpu.VMEM),
            pl.BlockSpec(memory_space=pltpu.MemorySpace.HBM),
            pl.BlockSpec(memory_space=pltpu.MemorySpace.HBM),
        ],
        out_specs=pl.BlockSpec(memory_space=pltpu.VMEM),
        scratch_shapes=[
            pltpu.VMEM((n_tok, 1), jnp.float32),
            pltpu.VMEM((n_local, d_model, d_out), jnp.float32),
            pltpu.VMEM((d_model, d_out), jnp.float32),
            pltpu.VMEM((2, N_DEV - 1, chunk, half), jnp.bfloat16),
            pltpu.VMEM((2, N_DEV - 1, chunk, half), jnp.bfloat16),
            pltpu.VMEM((2, chunk, half), jnp.bfloat16),
            pltpu.SemaphoreType.DMA((2,)),
            pltpu.SemaphoreType.DMA((2, 2, N_DEV - 1)),
            pltpu.SemaphoreType.DMA((2, 2, N_DEV - 1)),
        ],
        compiler_params=(
            pltpu.CompilerParams(collective_id=0)
            if _DO_BARRIER else pltpu.CompilerParams()
        ),
    )(x, router_W, route_idx, expert_W, shared_W)
